# baseline (speedup 1.0000x reference)
import functools

import jax
import jax.numpy as jnp
from jax import lax
from jax.experimental import pallas as pl
from jax.experimental.pallas import tpu as pltpu
from jax.experimental.pallas import tpu_sc as plsc

N = 10000
E = 160000
N_IN = 256
E_H = 512
N_H = 512
N_OUT = 256

_NC = 2
_NS = 16
_NW = _NC * _NS


def _sc_gather(x, src, dst, ne):
    mesh = plsc.VectorSubcoreMesh(core_axis_name="c", subcore_axis_name="s",
                                  num_cores=_NC, num_subcores=_NS)
    EW = ne // _NW
    CH = 40
    NCK = EW // CH

    @functools.partial(
        pl.kernel,
        out_type=[
            jax.ShapeDtypeStruct((ne, N_IN), jnp.float32),
            jax.ShapeDtypeStruct((ne, N_IN), jnp.float32),
        ],
        mesh=mesh,
        scratch_types=[
            pltpu.VMEM((EW,), jnp.int32),
            pltpu.VMEM((EW,), jnp.int32),
            pltpu.VMEM((CH, N_IN), jnp.float32),
            pltpu.VMEM((CH, N_IN), jnp.float32),
            pltpu.VMEM((CH, N_IN), jnp.float32),
            pltpu.VMEM((CH, N_IN), jnp.float32),
            pltpu.SemaphoreType.DMA,
            pltpu.SemaphoreType.DMA,
            pltpu.SemaphoreType.DMA,
            pltpu.SemaphoreType.DMA,
        ],
    )
    def k(x_hbm, src_hbm, dst_hbm, xs_hbm, xd_hbm,
          sidx, didx, a0, b0, a1, b1, s0a, s0b, s1a, s1b):
        wid = lax.axis_index("s") * _NC + lax.axis_index("c")
        base = wid * EW
        pltpu.sync_copy(src_hbm.at[pl.ds(base, EW)], sidx)
        pltpu.sync_copy(dst_hbm.at[pl.ds(base, EW)], didx)

        def start(ck, ab, bb, sa, sb):
            off = ck * CH
            pltpu.async_copy(x_hbm.at[sidx.at[pl.ds(off, CH)]], ab, sa)
            pltpu.async_copy(x_hbm.at[didx.at[pl.ds(off, CH)]], bb, sb)

        def wait(ab, bb, sa, sb):
            pltpu.make_async_copy(
                x_hbm.at[sidx.at[pl.ds(0, CH)]], ab, sa).wait()
            pltpu.make_async_copy(
                x_hbm.at[didx.at[pl.ds(0, CH)]], bb, sb).wait()

        def writeout(ck, ab, bb):
            off = ck * CH
            pltpu.sync_copy(ab, xs_hbm.at[pl.ds(base + off, CH)])
            pltpu.sync_copy(bb, xd_hbm.at[pl.ds(base + off, CH)])

        start(0, a0, b0, s0a, s0b)

        def body(j, carry):
            start(2 * j + 1, a1, b1, s1a, s1b)
            wait(a0, b0, s0a, s0b)
            writeout(2 * j, a0, b0)

            @pl.when(2 * j + 2 < NCK)
            def _():
                start(2 * j + 2, a0, b0, s0a, s0b)

            wait(a1, b1, s1a, s1b)
            writeout(2 * j + 1, a1, b1)
            return carry

        lax.fori_loop(0, NCK // 2, body, 0)
        if NCK % 2 == 1:
            wait(a0, b0, s0a, s0b)
            writeout(NCK - 1, a0, b0)

    return k(x, src, dst)


def _edge_body(xs_ref, xd_ref, wes_ref, wed_ref, w1x_ref, w1b_ref,
               be_ref, b1_ref, h_ref):
    xs = xs_ref[...].astype(jnp.bfloat16)
    xd = xd_ref[...].astype(jnp.bfloat16)
    epre = jnp.dot(xs, wes_ref[...], preferred_element_type=jnp.float32)
    epre = epre + jnp.dot(xd, wed_ref[...], preferred_element_type=jnp.float32)
    e = jnp.maximum(epre + be_ref[...], 0.0).astype(jnp.bfloat16)
    hpre = jnp.dot(e, w1b_ref[...], preferred_element_type=jnp.float32)
    hpre = hpre + jnp.dot(xd, w1x_ref[...], preferred_element_type=jnp.float32)
    h = jnp.maximum(hpre + b1_ref[...], 0.0)
    for q in range(4):
        h_ref[q] = h[:, 128 * q:128 * (q + 1)]


def _edge_mlp(xs, xd, wes, wed, w1x, w1b, be2, b12, ne):
    BE = 1280
    return pl.pallas_call(
        _edge_body,
        grid=(ne // BE,),
        in_specs=[
            pl.BlockSpec((BE, N_IN), lambda i: (i, 0)),
            pl.BlockSpec((BE, N_IN), lambda i: (i, 0)),
            pl.BlockSpec((N_IN, E_H), lambda i: (0, 0)),
            pl.BlockSpec((N_IN, E_H), lambda i: (0, 0)),
            pl.BlockSpec((N_IN, N_H), lambda i: (0, 0)),
            pl.BlockSpec((E_H, N_H), lambda i: (0, 0)),
            pl.BlockSpec((1, E_H), lambda i: (0, 0)),
            pl.BlockSpec((1, N_H), lambda i: (0, 0)),
        ],
        out_specs=pl.BlockSpec((4, BE, 128), lambda i: (0, i, 0)),
        out_shape=jax.ShapeDtypeStruct((4, ne, 128), jnp.float32),
    )(xs, xd, wes, wed, w1x, w1b, be2, b12)


def _sc_scatter(hs, dsts, nes):
    mesh = plsc.VectorSubcoreMesh(core_axis_name="c", subcore_axis_name="s",
                                  num_cores=_NC, num_subcores=_NS)
    CH = 80
    NB = 640
    nseg = len(hs)

    @functools.partial(
        pl.kernel,
        out_type=jax.ShapeDtypeStruct((4, N, 128), jnp.float32),
        mesh=mesh,
        scratch_types=[
            pltpu.VMEM((CH,), jnp.int32),
            pltpu.VMEM((CH,), jnp.int32),
            pltpu.VMEM((CH, 128), jnp.float32),
            pltpu.VMEM((CH, 128), jnp.float32),
            pltpu.VMEM_SHARED((N, 128), jnp.float32),
            pltpu.SemaphoreType.DMA,
            pltpu.SemaphoreType.DMA,
            pltpu.SemaphoreType.DMA,
            pltpu.SemaphoreType.DMA,
        ],
    )
    def k(*args):
        h_hbms = args[:nseg]
        dst_hbms = args[nseg:2 * nseg]
        agg_hbm = args[2 * nseg]
        (idx0, idx1, h0, h1, acc, si0, sh0, si1, sh1) = args[2 * nseg + 1:]
        c = lax.axis_index("c")
        s = lax.axis_index("s")

        for q in range(2):
            qidx = c * 2 + q

            def zrow(r, carry):
                for cc in range(8):
                    h0[r, pl.ds(cc * 16, 16)] = jnp.zeros((16,), jnp.float32)
                return carry

            lax.fori_loop(0, CH, zrow, 0)

            @pl.when(s < 15)
            def _():
                for j in range(NB // CH):
                    pltpu.sync_copy(h0, acc.at[pl.ds(s * NB + j * CH, CH)])

            @pl.when(s == 15)
            def _():
                for j in range((N - 15 * NB) // CH):
                    pltpu.sync_copy(h0, acc.at[pl.ds(15 * NB + j * CH, CH)])

            plsc.subcore_barrier()

            for h_hbm, dst_hbm, ne in zip(h_hbms, dst_hbms, nes):
                ES = ne // _NS
                NCK = ES // CH

                def start(ck, ib, hb, si, sh):
                    eb = s * ES + ck * CH
                    pltpu.async_copy(dst_hbm.at[pl.ds(eb, CH)], ib, si)
                    pltpu.async_copy(h_hbm.at[qidx, pl.ds(eb, CH)], hb, sh)

                def wait(ib, hb, si, sh):
                    pltpu.make_async_copy(
                        dst_hbm.at[pl.ds(0, CH)], ib, si).wait()
                    pltpu.make_async_copy(
                        h_hbm.at[qidx, pl.ds(0, CH)], hb, sh).wait()

                def scat(ib, hb):
                    pltpu.sync_copy(hb, acc.at[ib], add=True)

                start(0, idx0, h0, si0, sh0)

                def body(j, carry):
                    start(2 * j + 1, idx1, h1, si1, sh1)
                    wait(idx0, h0, si0, sh0)
                    scat(idx0, h0)

                    @pl.when(2 * j + 2 < NCK)
                    def _():
                        start(2 * j + 2, idx0, h0, si0, sh0)

                    wait(idx1, h1, si1, sh1)
                    scat(idx1, h1)
                    return carry

                lax.fori_loop(0, NCK // 2, body, 0)
                if NCK % 2 == 1:
                    wait(idx0, h0, si0, sh0)
                    scat(idx0, h0)

            plsc.subcore_barrier()

            @pl.when(s < 15)
            def _():
                pltpu.sync_copy(acc.at[pl.ds(s * NB, NB)],
                                agg_hbm.at[qidx, pl.ds(s * NB, NB)])

            @pl.when(s == 15)
            def _():
                pltpu.sync_copy(acc.at[pl.ds(15 * NB, N - 15 * NB)],
                                agg_hbm.at[qidx, pl.ds(15 * NB, N - 15 * NB)])

            plsc.subcore_barrier()

    return k(*hs, *dsts)


def _out_body(a_ref, b4_ref, w_ref, b_ref, o_ref):
    a = a_ref[...]
    b = b4_ref[...]
    acc = None
    for q in range(4):
        t = a[q] + b[q]
        p = jnp.dot(t, w_ref[q], preferred_element_type=jnp.float32)
        acc = p if acc is None else acc + p
    o_ref[...] = jnp.maximum(acc + b_ref[...], 0.0)


def _final(agg4a, agg4b, w2r, b2r):
    BN = 1000
    return pl.pallas_call(
        _out_body,
        grid=(N // BN,),
        in_specs=[
            pl.BlockSpec((4, BN, 128), lambda i: (0, i, 0)),
            pl.BlockSpec((4, BN, 128), lambda i: (0, i, 0)),
            pl.BlockSpec((4, 128, N_OUT), lambda i: (0, 0, 0)),
            pl.BlockSpec((1, N_OUT), lambda i: (0, 0)),
        ],
        out_specs=pl.BlockSpec((BN, N_OUT), lambda i: (i, 0)),
        out_shape=jax.ShapeDtypeStruct((N, N_OUT), jnp.float32),
    )(agg4a, agg4b, w2r, b2r)


_E1 = 64000
_E2 = 64000
_E3 = 16640
_E4 = E - _E1 - _E2 - _E3


def kernel(x, edge_index, We, be, W1, b1, W2, b2):
    src = edge_index[0]
    dst = edge_index[1]
    wes = We[:N_IN].astype(jnp.bfloat16)
    wed = We[N_IN:].astype(jnp.bfloat16)
    w1x = W1[:N_IN].astype(jnp.bfloat16)
    w1b = W1[N_IN:].astype(jnp.bfloat16)
    be2 = be.reshape(1, E_H)
    b12 = b1.reshape(1, N_H)
    s1, s2, s3 = _E1, _E1 + _E2, _E1 + _E2 + _E3
    xs1, xd1 = _sc_gather(x, src[:s1], dst[:s1], _E1)
    xs2, xd2 = _sc_gather(x, src[s1:s2], dst[s1:s2], _E2)
    xs3, xd3 = _sc_gather(x, src[s2:s3], dst[s2:s3], _E3)
    xs4, xd4 = _sc_gather(x, src[s3:], dst[s3:], _E4)
    h1 = _edge_mlp(xs1, xd1, wes, wed, w1x, w1b, be2, b12, _E1)
    h2 = _edge_mlp(xs2, xd2, wes, wed, w1x, w1b, be2, b12, _E2)
    agg12 = _sc_scatter([h1, h2], [dst[:s1], dst[s1:s2]], [_E1, _E2])
    h3 = _edge_mlp(xs3, xd3, wes, wed, w1x, w1b, be2, b12, _E3)
    h4 = _edge_mlp(xs4, xd4, wes, wed, w1x, w1b, be2, b12, _E4)
    agg34 = _sc_scatter([h3, h4], [dst[s2:s3], dst[s3:]], [_E3, _E4])
    return _final(agg12, agg34, W2.reshape(4, 128, N_OUT),
                  b2.reshape(1, N_OUT))


# speedup vs baseline: 1.0571x; 1.0571x over previous
import functools

import jax
import jax.numpy as jnp
from jax import lax
from jax.experimental import pallas as pl
from jax.experimental.pallas import tpu as pltpu
from jax.experimental.pallas import tpu_sc as plsc

N = 10000
E = 160000
N_IN = 256
E_H = 512
N_H = 512
N_OUT = 256

_NC = 2
_NS = 16
_NW = _NC * _NS


def _sc_gather(x, src, dst, ne):
    mesh = plsc.VectorSubcoreMesh(core_axis_name="c", subcore_axis_name="s",
                                  num_cores=_NC, num_subcores=_NS)
    EW = ne // _NW
    CH = 80 if EW % 80 == 0 else 40
    NCK = EW // CH

    @functools.partial(
        pl.kernel,
        out_type=[
            jax.ShapeDtypeStruct((ne, N_IN), jnp.float32),
            jax.ShapeDtypeStruct((ne, N_IN), jnp.float32),
        ],
        mesh=mesh,
        scratch_types=[
            pltpu.VMEM((EW,), jnp.int32),
            pltpu.VMEM((EW,), jnp.int32),
            pltpu.VMEM((CH, N_IN), jnp.float32),
            pltpu.VMEM((CH, N_IN), jnp.float32),
            pltpu.VMEM((CH, N_IN), jnp.float32),
            pltpu.VMEM((CH, N_IN), jnp.float32),
            pltpu.SemaphoreType.DMA,
            pltpu.SemaphoreType.DMA,
            pltpu.SemaphoreType.DMA,
            pltpu.SemaphoreType.DMA,
        ],
    )
    def k(x_hbm, src_hbm, dst_hbm, xs_hbm, xd_hbm,
          sidx, didx, a0, b0, a1, b1, s0a, s0b, s1a, s1b):
        wid = lax.axis_index("s") * _NC + lax.axis_index("c")
        base = wid * EW
        pltpu.sync_copy(src_hbm.at[pl.ds(base, EW)], sidx)
        pltpu.sync_copy(dst_hbm.at[pl.ds(base, EW)], didx)

        def start(ck, ab, bb, sa, sb):
            off = ck * CH
            pltpu.async_copy(x_hbm.at[sidx.at[pl.ds(off, CH)]], ab, sa)
            pltpu.async_copy(x_hbm.at[didx.at[pl.ds(off, CH)]], bb, sb)

        def wait(ab, bb, sa, sb):
            pltpu.make_async_copy(
                x_hbm.at[sidx.at[pl.ds(0, CH)]], ab, sa).wait()
            pltpu.make_async_copy(
                x_hbm.at[didx.at[pl.ds(0, CH)]], bb, sb).wait()

        def writeout(ck, ab, bb):
            off = ck * CH
            pltpu.sync_copy(ab, xs_hbm.at[pl.ds(base + off, CH)])
            pltpu.sync_copy(bb, xd_hbm.at[pl.ds(base + off, CH)])

        start(0, a0, b0, s0a, s0b)

        def body(j, carry):
            start(2 * j + 1, a1, b1, s1a, s1b)
            wait(a0, b0, s0a, s0b)
            writeout(2 * j, a0, b0)

            @pl.when(2 * j + 2 < NCK)
            def _():
                start(2 * j + 2, a0, b0, s0a, s0b)

            wait(a1, b1, s1a, s1b)
            writeout(2 * j + 1, a1, b1)
            return carry

        lax.fori_loop(0, NCK // 2, body, 0)
        if NCK % 2 == 1:
            wait(a0, b0, s0a, s0b)
            writeout(NCK - 1, a0, b0)

    return k(x, src, dst)


def _edge_body(xs_ref, xd_ref, wes_ref, wed_ref, w1x_ref, w1b_ref,
               be_ref, b1_ref, h_ref):
    xs = xs_ref[...].astype(jnp.bfloat16)
    xd = xd_ref[...].astype(jnp.bfloat16)
    epre = jnp.dot(xs, wes_ref[...], preferred_element_type=jnp.float32)
    epre = epre + jnp.dot(xd, wed_ref[...], preferred_element_type=jnp.float32)
    e = jnp.maximum(epre + be_ref[...], 0.0).astype(jnp.bfloat16)
    hpre = jnp.dot(e, w1b_ref[...], preferred_element_type=jnp.float32)
    hpre = hpre + jnp.dot(xd, w1x_ref[...], preferred_element_type=jnp.float32)
    h = jnp.maximum(hpre + b1_ref[...], 0.0)
    for q in range(4):
        h_ref[q] = h[:, 128 * q:128 * (q + 1)]


def _edge_mlp(xs, xd, wes, wed, w1x, w1b, be2, b12, ne):
    BE = 1280
    return pl.pallas_call(
        _edge_body,
        grid=(ne // BE,),
        in_specs=[
            pl.BlockSpec((BE, N_IN), lambda i: (i, 0)),
            pl.BlockSpec((BE, N_IN), lambda i: (i, 0)),
            pl.BlockSpec((N_IN, E_H), lambda i: (0, 0)),
            pl.BlockSpec((N_IN, E_H), lambda i: (0, 0)),
            pl.BlockSpec((N_IN, N_H), lambda i: (0, 0)),
            pl.BlockSpec((E_H, N_H), lambda i: (0, 0)),
            pl.BlockSpec((1, E_H), lambda i: (0, 0)),
            pl.BlockSpec((1, N_H), lambda i: (0, 0)),
        ],
        out_specs=pl.BlockSpec((4, BE, 128), lambda i: (0, i, 0)),
        out_shape=jax.ShapeDtypeStruct((4, ne, 128), jnp.float32),
    )(xs, xd, wes, wed, w1x, w1b, be2, b12)


def _sc_scatter(hs, dsts, nes):
    mesh = plsc.VectorSubcoreMesh(core_axis_name="c", subcore_axis_name="s",
                                  num_cores=_NC, num_subcores=_NS)
    CH = 80
    NB = 640
    nseg = len(hs)

    @functools.partial(
        pl.kernel,
        out_type=jax.ShapeDtypeStruct((4, N, 128), jnp.float32),
        mesh=mesh,
        scratch_types=[
            pltpu.VMEM((CH,), jnp.int32),
            pltpu.VMEM((CH,), jnp.int32),
            pltpu.VMEM((CH, 128), jnp.float32),
            pltpu.VMEM((CH, 128), jnp.float32),
            pltpu.VMEM_SHARED((N, 128), jnp.float32),
            pltpu.SemaphoreType.DMA,
            pltpu.SemaphoreType.DMA,
            pltpu.SemaphoreType.DMA,
            pltpu.SemaphoreType.DMA,
        ],
    )
    def k(*args):
        h_hbms = args[:nseg]
        dst_hbms = args[nseg:2 * nseg]
        agg_hbm = args[2 * nseg]
        (idx0, idx1, h0, h1, acc, si0, sh0, si1, sh1) = args[2 * nseg + 1:]
        c = lax.axis_index("c")
        s = lax.axis_index("s")

        for q in range(2):
            qidx = c * 2 + q

            def zrow(r, carry):
                for cc in range(8):
                    h0[r, pl.ds(cc * 16, 16)] = jnp.zeros((16,), jnp.float32)
                return carry

            lax.fori_loop(0, CH, zrow, 0)

            @pl.when(s < 15)
            def _():
                for j in range(NB // CH):
                    pltpu.sync_copy(h0, acc.at[pl.ds(s * NB + j * CH, CH)])

            @pl.when(s == 15)
            def _():
                for j in range((N - 15 * NB) // CH):
                    pltpu.sync_copy(h0, acc.at[pl.ds(15 * NB + j * CH, CH)])

            plsc.subcore_barrier()

            for h_hbm, dst_hbm, ne in zip(h_hbms, dst_hbms, nes):
                ES = ne // _NS
                NCK = ES // CH

                def start(ck, ib, hb, si, sh):
                    eb = s * ES + ck * CH
                    pltpu.async_copy(dst_hbm.at[pl.ds(eb, CH)], ib, si)
                    pltpu.async_copy(h_hbm.at[qidx, pl.ds(eb, CH)], hb, sh)

                def wait(ib, hb, si, sh):
                    pltpu.make_async_copy(
                        dst_hbm.at[pl.ds(0, CH)], ib, si).wait()
                    pltpu.make_async_copy(
                        h_hbm.at[qidx, pl.ds(0, CH)], hb, sh).wait()

                def scat(ib, hb):
                    pltpu.sync_copy(hb, acc.at[ib], add=True)

                start(0, idx0, h0, si0, sh0)

                def body(j, carry):
                    start(2 * j + 1, idx1, h1, si1, sh1)
                    wait(idx0, h0, si0, sh0)
                    scat(idx0, h0)

                    @pl.when(2 * j + 2 < NCK)
                    def _():
                        start(2 * j + 2, idx0, h0, si0, sh0)

                    wait(idx1, h1, si1, sh1)
                    scat(idx1, h1)
                    return carry

                lax.fori_loop(0, NCK // 2, body, 0)
                if NCK % 2 == 1:
                    wait(idx0, h0, si0, sh0)
                    scat(idx0, h0)

            plsc.subcore_barrier()

            @pl.when(s < 15)
            def _():
                pltpu.sync_copy(acc.at[pl.ds(s * NB, NB)],
                                agg_hbm.at[qidx, pl.ds(s * NB, NB)])

            @pl.when(s == 15)
            def _():
                pltpu.sync_copy(acc.at[pl.ds(15 * NB, N - 15 * NB)],
                                agg_hbm.at[qidx, pl.ds(15 * NB, N - 15 * NB)])

            plsc.subcore_barrier()

    return k(*hs, *dsts)


def _out_body(a_ref, b4_ref, w_ref, b_ref, o_ref):
    a = a_ref[...]
    b = b4_ref[...]
    acc = None
    for q in range(4):
        t = a[q] + b[q]
        p = jnp.dot(t, w_ref[q], preferred_element_type=jnp.float32)
        acc = p if acc is None else acc + p
    o_ref[...] = jnp.maximum(acc + b_ref[...], 0.0)


def _final(agg4a, agg4b, w2r, b2r):
    BN = 1000
    return pl.pallas_call(
        _out_body,
        grid=(N // BN,),
        in_specs=[
            pl.BlockSpec((4, BN, 128), lambda i: (0, i, 0)),
            pl.BlockSpec((4, BN, 128), lambda i: (0, i, 0)),
            pl.BlockSpec((4, 128, N_OUT), lambda i: (0, 0, 0)),
            pl.BlockSpec((1, N_OUT), lambda i: (0, 0)),
        ],
        out_specs=pl.BlockSpec((BN, N_OUT), lambda i: (i, 0)),
        out_shape=jax.ShapeDtypeStruct((N, N_OUT), jnp.float32),
    )(agg4a, agg4b, w2r, b2r)


_E1 = 51200
_E2 = 51200
_E3 = E - _E1 - _E2


def kernel(x, edge_index, We, be, W1, b1, W2, b2):
    src = edge_index[0]
    dst = edge_index[1]
    wes = We[:N_IN].astype(jnp.bfloat16)
    wed = We[N_IN:].astype(jnp.bfloat16)
    w1x = W1[:N_IN].astype(jnp.bfloat16)
    w1b = W1[N_IN:].astype(jnp.bfloat16)
    be2 = be.reshape(1, E_H)
    b12 = b1.reshape(1, N_H)
    s1, s2 = _E1, _E1 + _E2
    xs1, xd1 = _sc_gather(x, src[:s1], dst[:s1], _E1)
    xs2, xd2 = _sc_gather(x, src[s1:s2], dst[s1:s2], _E2)
    xs3, xd3 = _sc_gather(x, src[s2:], dst[s2:], _E3)
    h1 = _edge_mlp(xs1, xd1, wes, wed, w1x, w1b, be2, b12, _E1)
    h2 = _edge_mlp(xs2, xd2, wes, wed, w1x, w1b, be2, b12, _E2)
    agg12 = _sc_scatter([h1, h2], [dst[:s1], dst[s1:s2]], [_E1, _E2])
    h3 = _edge_mlp(xs3, xd3, wes, wed, w1x, w1b, be2, b12, _E3)
    agg3 = _sc_scatter([h3], [dst[s2:]], [_E3])
    return _final(agg12, agg3, W2.reshape(4, 128, N_OUT),
                  b2.reshape(1, N_OUT))
